# baseline (speedup 1.0000x reference)
import functools

import jax
import jax.numpy as jnp
from jax import lax
from jax.experimental import pallas as pl
from jax.experimental.pallas import tpu as pltpu
from jax.experimental.pallas import tpu_sc as plsc

N = 10000
NPAD = 10240
E = 320000
NSLAB = 2
HE = E // NSLAB
DP = 128
NC = 2
NS = 16
NW = NC * NS
PER_TILE = HE // NW
DMA_B = 125
CH = 1000
HCH = CH // 2
N_DMA = CH // DMA_B
N_DMA_H = HCH // DMA_B
N_CH = PER_TILE // CH
ROWS_PER_TILE = NPAD // NS

_f32 = jnp.float32
_bf16 = jnp.bfloat16


def _sc_mesh():
    return plsc.VectorSubcoreMesh(core_axis_name="c", subcore_axis_name="s")


_SC_PARAMS = pltpu.CompilerParams(use_tc_tiling_on_sc=False)


def _sc_gather(xpadb, row2, half):
    @functools.partial(
        pl.kernel,
        mesh=_sc_mesh(),
        compiler_params=_SC_PARAMS,
        out_type=jax.ShapeDtypeStruct((HE, DP), _bf16),
        scratch_types=[
            pltpu.VMEM((N_DMA, DMA_B), jnp.int32),
            pltpu.VMEM((CH, DP), _bf16),
            pltpu.SemaphoreType.DMA,
        ],
    )
    def k(x_hbm, row_hbm, out_hbm, idx_v, rows_v, sem):
        cid = lax.axis_index("c")
        sid = lax.axis_index("s")
        wid = sid * NC + cid

        def chunk(i, carry):
            ebase = pl.multiple_of(wid * PER_TILE + i * CH, CH)
            rbase = pl.multiple_of(
                half * (HE // DMA_B) + wid * (PER_TILE // DMA_B) + i * N_DMA,
                N_DMA)
            pltpu.sync_copy(row_hbm.at[pl.ds(rbase, N_DMA)], idx_v)
            descs = []
            for j in range(N_DMA):
                descs.append(
                    pltpu.async_copy(
                        x_hbm.at[idx_v.at[j]],
                        rows_v.at[pl.ds(j * DMA_B, DMA_B)],
                        sem,
                    )
                )
            for d in descs:
                d.wait()
            pltpu.sync_copy(rows_v, out_hbm.at[pl.ds(ebase, CH)])
            return carry

        lax.fori_loop(0, N_CH, chunk, 0)

    return k(xpadb, row2)


def _sc_scatter(msg, col2, zeros_nd, half):
    @functools.partial(
        pl.kernel,
        mesh=_sc_mesh(),
        compiler_params=_SC_PARAMS,
        out_type=jax.ShapeDtypeStruct((NC * NPAD, 64), _f32),
        scratch_types=[
            pltpu.VMEM((N_DMA, DMA_B), jnp.int32),
            pltpu.VMEM((CH, 64), _f32),
            pltpu.VMEM_SHARED((NPAD, 64), _f32),
            pltpu.SemaphoreType.DMA,
        ],
    )
    def k(msg_hbm, col_hbm, z_hbm, out_hbm, idx_v, vals_v, shared, sem):
        cid = lax.axis_index("c")
        sid = lax.axis_index("s")
        wid = sid * NC + cid
        myrow = pl.multiple_of(sid * ROWS_PER_TILE, ROWS_PER_TILE)

        pltpu.sync_copy(
            z_hbm.at[pl.ds(myrow, ROWS_PER_TILE)],
            shared.at[pl.ds(myrow, ROWS_PER_TILE)],
        )
        plsc.subcore_barrier()

        def chunk(i, carry):
            ebase = pl.multiple_of(wid * PER_TILE + i * CH, CH)
            rbase = pl.multiple_of(
                half * (HE // DMA_B) + wid * (PER_TILE // DMA_B) + i * N_DMA,
                N_DMA)
            pltpu.sync_copy(col_hbm.at[pl.ds(rbase, N_DMA)], idx_v)
            pltpu.sync_copy(msg_hbm.at[pl.ds(ebase, CH), pl.ds(0, 64)], vals_v)
            for j in range(N_DMA):
                pltpu.sync_copy(
                    vals_v.at[pl.ds(j * DMA_B, DMA_B)],
                    shared.at[idx_v.at[j]],
                    add=True,
                )
            return carry

        lax.fori_loop(0, N_CH, chunk, 0)
        plsc.subcore_barrier()
        pltpu.sync_copy(
            shared.at[pl.ds(myrow, ROWS_PER_TILE)],
            out_hbm.at[pl.ds(cid * NPAD + myrow, ROWS_PER_TILE)],
        )

    return k(msg, col2, zeros_nd)


_BE = 3200


def _tc_edge_mlp(xg, ea, wx, we, w1, b1, w2, b2, w3, b3, half):
    def body(xg_ref, ea_ref, wx_ref, we_ref, w1_ref, b1_ref,
             w2_ref, b2_ref, w3_ref, b3_ref, out_ref):
        xgb = xg_ref[...]
        eab = ea_ref[...].astype(_bf16)
        h = jnp.dot(xgb, wx_ref[...], preferred_element_type=_f32)
        h = h + jnp.dot(eab, we_ref[...], preferred_element_type=_f32)
        h = jnp.maximum(h, 0.0).astype(_bf16)
        h = jnp.dot(h, w1_ref[...], preferred_element_type=_f32) + b1_ref[...]
        h = jnp.maximum(h, 0.0).astype(_bf16)
        h = jnp.dot(h, w2_ref[...], preferred_element_type=_f32) + b2_ref[...]
        h = jnp.maximum(h, 0.0).astype(_bf16)
        out_ref[...] = (
            jnp.dot(h, w3_ref[...], preferred_element_type=_f32) + b3_ref[...]
        )

    def full(shape):
        return pl.BlockSpec(shape, lambda i: (0, 0))

    off = half * (HE // _BE)
    return pl.pallas_call(
        body,
        grid=(HE // _BE,),
        in_specs=[
            pl.BlockSpec((_BE, DP), lambda i: (i, 0)),
            pl.BlockSpec((_BE, 50), lambda i: (i + off, 0)),
            full((DP, 1024)), full((50, 1024)),
            full((1024, 512)), full((1, 512)),
            full((512, 128)), full((1, 128)),
            full((128, DP)), full((1, DP)),
        ],
        out_specs=pl.BlockSpec((_BE, DP), lambda i: (i, 0)),
        out_shape=jax.ShapeDtypeStruct((HE, DP), _f32),
    )(xg, ea, wx, we, w1, b1, w2, b2, w3, b3)


_BN = 2000


def _tc_node_mlp(pa, pb, xpadb, wx, wa, w1, b1, w2, b2, w3, b3):
    def body(a0_ref, a1_ref, b0_ref, b1p_ref, x_ref, wx_ref, wa_ref,
             w1_ref, b1_ref, w2_ref, b2_ref, w3_ref, b3_ref, out_ref):
        s = (a0_ref[0] + a1_ref[0]) + (b0_ref[0] + b1p_ref[0])
        cnt = jnp.maximum(s[:, 50:51], 1.0)
        agg = (s / cnt).astype(_bf16)
        xb = x_ref[...]
        h = jnp.dot(xb, wx_ref[...], preferred_element_type=_f32)
        h = h + jnp.dot(agg, wa_ref[...], preferred_element_type=_f32)
        h = jnp.maximum(h, 0.0).astype(_bf16)
        h = jnp.dot(h, w1_ref[...], preferred_element_type=_f32) + b1_ref[...]
        h = jnp.maximum(h, 0.0).astype(_bf16)
        h = jnp.dot(h, w2_ref[...], preferred_element_type=_f32) + b2_ref[...]
        h = jnp.maximum(h, 0.0).astype(_bf16)
        o = jnp.dot(h, w3_ref[...], preferred_element_type=_f32) + b3_ref[...]

        @pl.when(pl.program_id(0) == 0)
        def _init():
            out_ref[...] = jnp.zeros_like(out_ref)

        out_ref[...] += jnp.sum(o, axis=0, keepdims=True) * (1.0 / N)

    def full(shape):
        return pl.BlockSpec(shape, lambda i: (0, 0))

    part_spec0 = pl.BlockSpec((1, _BN, 64), lambda i: (0, i, 0))
    part_spec1 = pl.BlockSpec((1, _BN, 64), lambda i: (1, i, 0))
    return pl.pallas_call(
        body,
        grid=(N // _BN,),
        in_specs=[
            part_spec0, part_spec1, part_spec0, part_spec1,
            pl.BlockSpec((_BN, DP), lambda i: (i, 0)),
            full((DP, 1024)), full((64, 1024)),
            full((1024, 512)), full((1, 512)),
            full((512, 256)), full((1, 256)),
            full((256, 100)), full((1, 100)),
        ],
        out_specs=pl.BlockSpec((1, 100), lambda i: (0, 0)),
        out_shape=jax.ShapeDtypeStruct((1, 100), _f32),
    )(pa, pa, pb, pb, xpadb, wx, wa, w1, b1, w2, b2, w3, b3)


def kernel(x, edge_index, edge_attr,
           W1_0, b1_0, W1_1, b1_1, W1_2, b1_2, W1_3, b1_3,
           W2_0, b2_0, W2_1, b2_1, W2_2, b2_2, W2_3, b2_3):
    row2 = edge_index[0].astype(jnp.int32).reshape(E // DMA_B, DMA_B)
    col2 = edge_index[1].astype(jnp.int32).reshape(E // DMA_B, DMA_B)
    pad = DP - 50
    xpadb = jnp.pad(x, ((0, 0), (0, pad))).at[:, 50].set(1.0).astype(_bf16)

    wx = jnp.pad(W1_0[:50], ((0, pad), (0, 0))).at[50].set(b1_0).astype(_bf16)
    we = W1_0[50:].astype(_bf16)
    w1 = W1_1.astype(_bf16)
    b1 = b1_1.reshape(1, -1).astype(_bf16)
    w2 = W1_2.astype(_bf16)
    b2 = b1_2.reshape(1, -1).astype(_bf16)
    w3 = jnp.pad(W1_3, ((0, 0), (0, pad))).astype(_bf16)
    b3 = jnp.pad(b1_3, (0, pad)).at[50].set(1.0).reshape(1, -1)

    zeros_nd = jnp.zeros((NPAD, 64), _f32)

    xg0 = _sc_gather(xpadb, row2, 0)
    msg0 = _tc_edge_mlp(xg0, edge_attr, wx, we, w1, b1, w2, b2, w3, b3, 0)
    xg1 = _sc_gather(xpadb, row2, 1)
    msg1 = _tc_edge_mlp(xg1, edge_attr, wx, we, w1, b1, w2, b2, w3, b3, 1)
    pa = _sc_scatter(msg0, col2, zeros_nd, 0).reshape(NC, NPAD, 64)
    pb = _sc_scatter(msg1, col2, zeros_nd, 1).reshape(NC, NPAD, 64)

    ux = jnp.pad(W2_0[:50], ((0, pad), (0, 0))).at[50].set(b2_0).astype(_bf16)
    ua = jnp.pad(W2_0[50:], ((0, 14), (0, 0))).astype(_bf16)
    u1 = W2_1.astype(_bf16)
    d1 = b2_1.reshape(1, -1)
    u2 = W2_2.astype(_bf16)
    d2 = b2_2.reshape(1, -1)
    u3 = W2_3.astype(_bf16)
    d3 = b2_3.reshape(1, -1)

    return _tc_node_mlp(pa, pb, xpadb, ux, ua, u1, d1, u2, d2, u3, d3)


# speedup vs baseline: 1.4106x; 1.4106x over previous
import functools

import jax
import jax.numpy as jnp
from jax import lax
from jax.experimental import pallas as pl
from jax.experimental.pallas import tpu as pltpu
from jax.experimental.pallas import tpu_sc as plsc

N = 10000
NPAD = 10240
E = 320000
NSLAB = 2
HE = E // NSLAB
DP = 128
NC = 2
NS = 16
NW = NC * NS
PER_TILE = HE // NW
DMA_B = 125
CH = 1000
HCH = CH // 2
N_DMA = CH // DMA_B
N_DMA_H = HCH // DMA_B
N_CH = PER_TILE // CH
ROWS_PER_TILE = NPAD // NS

_f32 = jnp.float32
_bf16 = jnp.bfloat16


def _sc_mesh():
    return plsc.VectorSubcoreMesh(core_axis_name="c", subcore_axis_name="s")


_SC_PARAMS = pltpu.CompilerParams(use_tc_tiling_on_sc=False)


def _sc_gather(xpadb, row2, half):
    @functools.partial(
        pl.kernel,
        mesh=_sc_mesh(),
        compiler_params=_SC_PARAMS,
        out_type=jax.ShapeDtypeStruct((HE, DP), _f32),
        scratch_types=[
            pltpu.VMEM((N_DMA, DMA_B), jnp.int32),
            pltpu.VMEM((CH, DP), _f32),
            pltpu.SemaphoreType.DMA,
        ],
    )
    def k(x_hbm, row_hbm, out_hbm, idx_v, rows_v, sem):
        cid = lax.axis_index("c")
        sid = lax.axis_index("s")
        wid = sid * NC + cid

        def chunk(i, carry):
            ebase = pl.multiple_of(wid * PER_TILE + i * CH, CH)
            rbase = pl.multiple_of(
                half * (HE // DMA_B) + wid * (PER_TILE // DMA_B) + i * N_DMA,
                N_DMA)
            pltpu.sync_copy(row_hbm.at[pl.ds(rbase, N_DMA)], idx_v)
            descs = []
            for j in range(N_DMA):
                descs.append(
                    pltpu.async_copy(
                        x_hbm.at[idx_v.at[j]],
                        rows_v.at[pl.ds(j * DMA_B, DMA_B)],
                        sem,
                    )
                )
            for d in descs:
                d.wait()
            pltpu.sync_copy(rows_v, out_hbm.at[pl.ds(ebase, CH)])
            return carry

        lax.fori_loop(0, N_CH, chunk, 0)

    return k(xpadb, row2)


def _sc_scatter(msg, col2, zeros_nd, half):
    @functools.partial(
        pl.kernel,
        mesh=_sc_mesh(),
        compiler_params=_SC_PARAMS,
        out_type=jax.ShapeDtypeStruct((NC * NPAD, 64), _f32),
        scratch_types=[
            pltpu.VMEM((N_DMA, DMA_B), jnp.int32),
            pltpu.VMEM((CH, 64), _f32),
            pltpu.VMEM_SHARED((NPAD, 64), _f32),
            pltpu.SemaphoreType.DMA,
        ],
    )
    def k(msg_hbm, col_hbm, z_hbm, out_hbm, idx_v, vals_v, shared, sem):
        cid = lax.axis_index("c")
        sid = lax.axis_index("s")
        wid = sid * NC + cid
        myrow = pl.multiple_of(sid * ROWS_PER_TILE, ROWS_PER_TILE)

        pltpu.sync_copy(
            z_hbm.at[pl.ds(myrow, ROWS_PER_TILE)],
            shared.at[pl.ds(myrow, ROWS_PER_TILE)],
        )
        plsc.subcore_barrier()

        def chunk(i, carry):
            ebase = pl.multiple_of(wid * PER_TILE + i * CH, CH)
            rbase = pl.multiple_of(
                half * (HE // DMA_B) + wid * (PER_TILE // DMA_B) + i * N_DMA,
                N_DMA)
            pltpu.sync_copy(col_hbm.at[pl.ds(rbase, N_DMA)], idx_v)
            pltpu.sync_copy(msg_hbm.at[pl.ds(ebase, CH), pl.ds(0, 64)], vals_v)
            for j in range(N_DMA):
                pltpu.sync_copy(
                    vals_v.at[pl.ds(j * DMA_B, DMA_B)],
                    shared.at[idx_v.at[j]],
                    add=True,
                )
            return carry

        lax.fori_loop(0, N_CH, chunk, 0)
        plsc.subcore_barrier()
        pltpu.sync_copy(
            shared.at[pl.ds(myrow, ROWS_PER_TILE)],
            out_hbm.at[pl.ds(cid * NPAD + myrow, ROWS_PER_TILE)],
        )

    return k(msg, col2, zeros_nd)


_BE = 3200


def _tc_edge_mlp(xg, ea, wx, we, w1, b1, w2, b2, w3, b3, half):
    def body(xg_ref, ea_ref, wx_ref, we_ref, w1_ref, b1_ref,
             w2_ref, b2_ref, w3_ref, b3_ref, out_ref):
        xgb = xg_ref[...].astype(_bf16)
        eab = ea_ref[...].astype(_bf16)
        h = jnp.dot(xgb, wx_ref[...], preferred_element_type=_f32)
        h = h + lax.dot_general(eab, we_ref[...],
                                (((0,), (0,)), ((), ())),
                                preferred_element_type=_f32)
        h = jnp.maximum(h, 0.0).astype(_bf16)
        h = jnp.dot(h, w1_ref[...], preferred_element_type=_f32) + b1_ref[...]
        h = jnp.maximum(h, 0.0).astype(_bf16)
        h = jnp.dot(h, w2_ref[...], preferred_element_type=_f32) + b2_ref[...]
        h = jnp.maximum(h, 0.0).astype(_bf16)
        out_ref[...] = (
            jnp.dot(h, w3_ref[...], preferred_element_type=_f32) + b3_ref[...]
        )

    def full(shape):
        return pl.BlockSpec(shape, lambda i: (0, 0))

    off = half * (HE // _BE)
    return pl.pallas_call(
        body,
        grid=(HE // _BE,),
        in_specs=[
            pl.BlockSpec((_BE, DP), lambda i: (i, 0)),
            pl.BlockSpec((50, _BE), lambda i: (0, i + off)),
            full((DP, 1024)), full((50, 1024)),
            full((1024, 512)), full((1, 512)),
            full((512, 128)), full((1, 128)),
            full((128, DP)), full((1, DP)),
        ],
        out_specs=pl.BlockSpec((_BE, DP), lambda i: (i, 0)),
        out_shape=jax.ShapeDtypeStruct((HE, DP), _f32),
    )(xg, ea, wx, we, w1, b1, w2, b2, w3, b3)


_BN = 2000


def _tc_node_mlp(pa, pb, xpadb, wx, wa, w1, b1, w2, b2, w3, b3):
    def body(a0_ref, a1_ref, b0_ref, b1p_ref, x_ref, wx_ref, wa_ref,
             w1_ref, b1_ref, w2_ref, b2_ref, w3_ref, b3_ref, out_ref):
        s = (a0_ref[0] + a1_ref[0]) + (b0_ref[0] + b1p_ref[0])
        cnt = jnp.maximum(s[:, 50:51], 1.0)
        agg = (s / cnt).astype(_bf16)
        xb = x_ref[...].astype(_bf16)
        h = jnp.dot(xb, wx_ref[...], preferred_element_type=_f32)
        h = h + jnp.dot(agg, wa_ref[...], preferred_element_type=_f32)
        h = jnp.maximum(h, 0.0).astype(_bf16)
        h = jnp.dot(h, w1_ref[...], preferred_element_type=_f32) + b1_ref[...]
        h = jnp.maximum(h, 0.0).astype(_bf16)
        h = jnp.dot(h, w2_ref[...], preferred_element_type=_f32) + b2_ref[...]
        h = jnp.maximum(h, 0.0).astype(_bf16)
        o = jnp.dot(h, w3_ref[...], preferred_element_type=_f32) + b3_ref[...]

        @pl.when(pl.program_id(0) == 0)
        def _init():
            out_ref[...] = jnp.zeros_like(out_ref)

        out_ref[...] += jnp.sum(o, axis=0, keepdims=True) * (1.0 / N)

    def full(shape):
        return pl.BlockSpec(shape, lambda i: (0, 0))

    part_spec0 = pl.BlockSpec((1, _BN, 64), lambda i: (0, i, 0))
    part_spec1 = pl.BlockSpec((1, _BN, 64), lambda i: (1, i, 0))
    return pl.pallas_call(
        body,
        grid=(N // _BN,),
        in_specs=[
            part_spec0, part_spec1, part_spec0, part_spec1,
            pl.BlockSpec((_BN, DP), lambda i: (i, 0)),
            full((DP, 1024)), full((64, 1024)),
            full((1024, 512)), full((1, 512)),
            full((512, 256)), full((1, 256)),
            full((256, 100)), full((1, 100)),
        ],
        out_specs=pl.BlockSpec((1, 100), lambda i: (0, 0)),
        out_shape=jax.ShapeDtypeStruct((1, 100), _f32),
    )(pa, pa, pb, pb, xpadb, wx, wa, w1, b1, w2, b2, w3, b3)


def kernel(x, edge_index, edge_attr,
           W1_0, b1_0, W1_1, b1_1, W1_2, b1_2, W1_3, b1_3,
           W2_0, b2_0, W2_1, b2_1, W2_2, b2_2, W2_3, b2_3):
    row2 = edge_index[0].astype(jnp.int32).reshape(E // DMA_B, DMA_B)
    col2 = edge_index[1].astype(jnp.int32).reshape(E // DMA_B, DMA_B)
    pad = DP - 50
    xpadf = jnp.pad(x, ((0, 0), (0, pad))).at[:, 50].set(1.0)

    wx = jnp.pad(W1_0[:50], ((0, pad), (0, 0))).at[50].set(b1_0).astype(_bf16)
    we = W1_0[50:].astype(_bf16)
    w1 = W1_1.astype(_bf16)
    b1 = b1_1.reshape(1, -1).astype(_bf16)
    w2 = W1_2.astype(_bf16)
    b2 = b1_2.reshape(1, -1).astype(_bf16)
    w3 = jnp.pad(W1_3, ((0, 0), (0, pad))).astype(_bf16)
    b3 = jnp.pad(b1_3, (0, pad)).at[50].set(1.0).reshape(1, -1)

    zeros_nd = jnp.zeros((NPAD, 64), _f32)

    eat = edge_attr.T
    xg0 = _sc_gather(xpadf, row2, 0)
    msg0 = _tc_edge_mlp(xg0, eat, wx, we, w1, b1, w2, b2, w3, b3, 0)
    xg1 = _sc_gather(xpadf, row2, 1)
    msg1 = _tc_edge_mlp(xg1, eat, wx, we, w1, b1, w2, b2, w3, b3, 1)
    pa = _sc_scatter(msg0, col2, zeros_nd, 0).reshape(NC, NPAD, 64)
    pb = _sc_scatter(msg1, col2, zeros_nd, 1).reshape(NC, NPAD, 64)

    ux = jnp.pad(W2_0[:50], ((0, pad), (0, 0))).at[50].set(b2_0).astype(_bf16)
    ua = jnp.pad(W2_0[50:], ((0, 14), (0, 0))).astype(_bf16)
    u1 = W2_1.astype(_bf16)
    d1 = b2_1.reshape(1, -1)
    u2 = W2_2.astype(_bf16)
    d2 = b2_2.reshape(1, -1)
    u3 = W2_3.astype(_bf16)
    d3 = b2_3.reshape(1, -1)

    return _tc_node_mlp(pa, pb, xpadf, ux, ua, u1, d1, u2, d2, u3, d3)


# speedup vs baseline: 1.4801x; 1.0493x over previous
import functools

import jax
import jax.numpy as jnp
from jax import lax
from jax.experimental import pallas as pl
from jax.experimental.pallas import tpu as pltpu
from jax.experimental.pallas import tpu_sc as plsc

N = 10000
NPAD = 10240
E = 320000
NSLAB = 4
HE = E // NSLAB
DP = 128
NC = 2
NS = 16
NW = NC * NS
PER_TILE = HE // NW
DMA_B = 125
CH = 500
HCH = CH // 2
N_DMA = CH // DMA_B
N_DMA_H = HCH // DMA_B
N_CH = PER_TILE // CH
ROWS_PER_TILE = NPAD // NS

_f32 = jnp.float32
_bf16 = jnp.bfloat16


def _sc_mesh():
    return plsc.VectorSubcoreMesh(core_axis_name="c", subcore_axis_name="s")


_SC_PARAMS = pltpu.CompilerParams(use_tc_tiling_on_sc=False)


def _sc_gather(xpadb, row2):
    @functools.partial(
        pl.kernel,
        mesh=_sc_mesh(),
        compiler_params=_SC_PARAMS,
        out_type=jax.ShapeDtypeStruct((HE, DP), _f32),
        scratch_types=[
            pltpu.VMEM((N_DMA, DMA_B), jnp.int32),
            pltpu.VMEM((CH, DP), _f32),
            pltpu.SemaphoreType.DMA,
        ],
    )
    def k(x_hbm, row_hbm, out_hbm, idx_v, rows_v, sem):
        cid = lax.axis_index("c")
        sid = lax.axis_index("s")
        wid = sid * NC + cid

        def chunk(i, carry):
            ebase = pl.multiple_of(wid * PER_TILE + i * CH, CH)
            rbase = pl.multiple_of(
                wid * (PER_TILE // DMA_B) + i * N_DMA, N_DMA)
            pltpu.sync_copy(row_hbm.at[pl.ds(rbase, N_DMA)], idx_v)
            descs = []
            for j in range(N_DMA):
                descs.append(
                    pltpu.async_copy(
                        x_hbm.at[idx_v.at[j]],
                        rows_v.at[pl.ds(j * DMA_B, DMA_B)],
                        sem,
                    )
                )
            for d in descs:
                d.wait()
            pltpu.sync_copy(rows_v, out_hbm.at[pl.ds(ebase, CH)])
            return carry

        lax.fori_loop(0, N_CH, chunk, 0)

    return k(xpadb, row2)


def _sc_scatter(msg, col2, zeros_nd):
    @functools.partial(
        pl.kernel,
        mesh=_sc_mesh(),
        compiler_params=_SC_PARAMS,
        out_type=jax.ShapeDtypeStruct((NC * NPAD, 64), _f32),
        scratch_types=[
            pltpu.VMEM((N_DMA, DMA_B), jnp.int32),
            pltpu.VMEM((CH, 64), _f32),
            pltpu.VMEM_SHARED((NPAD, 64), _f32),
            pltpu.SemaphoreType.DMA,
        ],
    )
    def k(msg_hbm, col_hbm, z_hbm, out_hbm, idx_v, vals_v, shared, sem):
        cid = lax.axis_index("c")
        sid = lax.axis_index("s")
        wid = sid * NC + cid
        myrow = pl.multiple_of(sid * ROWS_PER_TILE, ROWS_PER_TILE)

        pltpu.sync_copy(
            z_hbm.at[pl.ds(myrow, ROWS_PER_TILE)],
            shared.at[pl.ds(myrow, ROWS_PER_TILE)],
        )
        plsc.subcore_barrier()

        def chunk(i, carry):
            ebase = pl.multiple_of(wid * PER_TILE + i * CH, CH)
            rbase = pl.multiple_of(
                wid * (PER_TILE // DMA_B) + i * N_DMA, N_DMA)
            pltpu.sync_copy(col_hbm.at[pl.ds(rbase, N_DMA)], idx_v)
            pltpu.sync_copy(msg_hbm.at[pl.ds(ebase, CH), pl.ds(0, 64)], vals_v)
            for j in range(N_DMA):
                pltpu.sync_copy(
                    vals_v.at[pl.ds(j * DMA_B, DMA_B)],
                    shared.at[idx_v.at[j]],
                    add=True,
                )
            return carry

        lax.fori_loop(0, N_CH, chunk, 0)
        plsc.subcore_barrier()
        pltpu.sync_copy(
            shared.at[pl.ds(myrow, ROWS_PER_TILE)],
            out_hbm.at[pl.ds(cid * NPAD + myrow, ROWS_PER_TILE)],
        )

    return k(msg, col2, zeros_nd)


_BE = 3200


def _tc_edge_mlp(xg, ea, wx, we, w1, b1, w2, b2, w3, b3, half):
    def body(xg_ref, ea_ref, wx_ref, we_ref, w1_ref, b1_ref,
             w2_ref, b2_ref, w3_ref, b3_ref, out_ref):
        xgb = xg_ref[...].astype(_bf16)
        eab = ea_ref[...].astype(_bf16)
        h = jnp.dot(xgb, wx_ref[...], preferred_element_type=_f32)
        h = h + lax.dot_general(eab, we_ref[...],
                                (((0,), (0,)), ((), ())),
                                preferred_element_type=_f32)
        h = jnp.maximum(h, 0.0).astype(_bf16)
        h = jnp.dot(h, w1_ref[...], preferred_element_type=_f32) + b1_ref[...]
        h = jnp.maximum(h, 0.0).astype(_bf16)
        h = jnp.dot(h, w2_ref[...], preferred_element_type=_f32) + b2_ref[...]
        h = jnp.maximum(h, 0.0).astype(_bf16)
        out_ref[...] = (
            jnp.dot(h, w3_ref[...], preferred_element_type=_f32) + b3_ref[...]
        )

    def full(shape):
        return pl.BlockSpec(shape, lambda i: (0, 0))

    off = half * (HE // _BE)
    return pl.pallas_call(
        body,
        grid=(HE // _BE,),
        in_specs=[
            pl.BlockSpec((_BE, DP), lambda i: (i, 0)),
            pl.BlockSpec((50, _BE), lambda i: (0, i + off)),
            full((DP, 1024)), full((50, 1024)),
            full((1024, 512)), full((1, 512)),
            full((512, 128)), full((1, 128)),
            full((128, DP)), full((1, DP)),
        ],
        out_specs=pl.BlockSpec((_BE, DP), lambda i: (i, 0)),
        out_shape=jax.ShapeDtypeStruct((HE, DP), _f32),
    )(xg, ea, wx, we, w1, b1, w2, b2, w3, b3)


_BN = 2000


def _tc_node_mlp(parts, xpadb, wx, wa, w1, b1, w2, b2, w3, b3):
    def body(*refs):
        part_refs = refs[:2 * NSLAB]
        (x_ref, wx_ref, wa_ref, w1_ref, b1_ref, w2_ref, b2_ref,
         w3_ref, b3_ref, out_ref) = refs[2 * NSLAB:]
        s = part_refs[0][0]
        for pr in part_refs[1:]:
            s = s + pr[0]
        cnt = jnp.maximum(s[:, 50:51], 1.0)
        agg = (s / cnt).astype(_bf16)
        xb = x_ref[...].astype(_bf16)
        h = jnp.dot(xb, wx_ref[...], preferred_element_type=_f32)
        h = h + jnp.dot(agg, wa_ref[...], preferred_element_type=_f32)
        h = jnp.maximum(h, 0.0).astype(_bf16)
        h = jnp.dot(h, w1_ref[...], preferred_element_type=_f32) + b1_ref[...]
        h = jnp.maximum(h, 0.0).astype(_bf16)
        h = jnp.dot(h, w2_ref[...], preferred_element_type=_f32) + b2_ref[...]
        h = jnp.maximum(h, 0.0).astype(_bf16)
        o = jnp.dot(h, w3_ref[...], preferred_element_type=_f32) + b3_ref[...]

        @pl.when(pl.program_id(0) == 0)
        def _init():
            out_ref[...] = jnp.zeros_like(out_ref)

        out_ref[...] += jnp.sum(o, axis=0, keepdims=True) * (1.0 / N)

    def full(shape):
        return pl.BlockSpec(shape, lambda i: (0, 0))

    part_spec0 = pl.BlockSpec((1, _BN, 64), lambda i: (0, i, 0))
    part_spec1 = pl.BlockSpec((1, _BN, 64), lambda i: (1, i, 0))
    part_specs = [part_spec0, part_spec1] * NSLAB
    part_args = [p for p in parts for _ in range(2)]
    return pl.pallas_call(
        body,
        grid=(N // _BN,),
        in_specs=[
            *part_specs,
            pl.BlockSpec((_BN, DP), lambda i: (i, 0)),
            full((DP, 1024)), full((64, 1024)),
            full((1024, 512)), full((1, 512)),
            full((512, 256)), full((1, 256)),
            full((256, 100)), full((1, 100)),
        ],
        out_specs=pl.BlockSpec((1, 100), lambda i: (0, 0)),
        out_shape=jax.ShapeDtypeStruct((1, 100), _f32),
    )(*part_args, xpadb, wx, wa, w1, b1, w2, b2, w3, b3)


def kernel(x, edge_index, edge_attr,
           W1_0, b1_0, W1_1, b1_1, W1_2, b1_2, W1_3, b1_3,
           W2_0, b2_0, W2_1, b2_1, W2_2, b2_2, W2_3, b2_3):
    row2 = edge_index[0].astype(jnp.int32).reshape(E // DMA_B, DMA_B)
    col2 = edge_index[1].astype(jnp.int32).reshape(E // DMA_B, DMA_B)
    pad = DP - 50
    xpadf = jnp.pad(jnp.pad(x, ((0, 0), (0, 1)), constant_values=1.0),
                    ((0, 0), (0, pad - 1)))

    wx = jnp.pad(W1_0[:50], ((0, pad), (0, 0))).at[50].set(b1_0).astype(_bf16)
    we = W1_0[50:].astype(_bf16)
    w1 = W1_1.astype(_bf16)
    b1 = b1_1.reshape(1, -1).astype(_bf16)
    w2 = W1_2.astype(_bf16)
    b2 = b1_2.reshape(1, -1).astype(_bf16)
    w3 = jnp.pad(W1_3, ((0, 0), (0, pad))).astype(_bf16)
    b3 = jnp.pad(b1_3, (0, pad)).at[50].set(1.0).reshape(1, -1)

    zeros_nd = jnp.zeros((NPAD, 64), _f32)

    eat = edge_attr.T
    rps = HE // DMA_B
    msgs = []
    for k in range(NSLAB):
        xg_k = _sc_gather(xpadf, row2[k * rps:(k + 1) * rps])
        msgs.append(_tc_edge_mlp(xg_k, eat, wx, we, w1, b1, w2, b2, w3, b3, k))
    parts = [
        _sc_scatter(msgs[k], col2[k * rps:(k + 1) * rps],
                    zeros_nd).reshape(NC, NPAD, 64)
        for k in range(NSLAB)
    ]

    ux = jnp.pad(W2_0[:50], ((0, pad), (0, 0))).at[50].set(b2_0).astype(_bf16)
    ua = jnp.pad(W2_0[50:], ((0, 14), (0, 0))).astype(_bf16)
    u1 = W2_1.astype(_bf16)
    d1 = b2_1.reshape(1, -1)
    u2 = W2_2.astype(_bf16)
    d2 = b2_2.reshape(1, -1)
    u3 = W2_3.astype(_bf16)
    d3 = b2_3.reshape(1, -1)

    return _tc_node_mlp(parts, xpadf, ux, ua, u1, d1, u2, d2, u3, d3)
